# baseline (speedup 1.0000x reference)
import functools
import math

import jax
import jax.numpy as jnp
from jax import lax
from jax.experimental import pallas as pl
from jax.experimental.pallas import tpu as pltpu
from jax.experimental.pallas import tpu_sc as plsc

N = 10000
E = 320000
IN_CH = 128
HID = 128
OUT_CH = 40
NUM_LAYERS = 4

NC = 2
NS = 16
NW = NC * NS
NP = 10240
RPS = NP // NS
CH = 32
EPT = NP
CPT = EPT // CH
PAD = EPT - E // NW
DCH = 64
DCPT = EPT // DCH
HR = NP // 2
HRS = HR // NS
ZROW = 10000
TROWS = 10001
TRS = 632
TLAST = N - (NS - 1) * TRS

BLK = 512
GRID = NP // BLK

_C = math.sqrt(math.pi) / 2.0

_mesh = plsc.VectorSubcoreMesh(core_axis_name="c", subcore_axis_name="s")


DEPTH = 2
IRING = 6
IP = 4
DDEPTH = 4


@functools.partial(
    pl.kernel,
    out_type=jax.ShapeDtypeStruct((NC, NP, HID), jnp.float32),
    mesh=_mesh,
    scratch_types=[
        pltpu.VMEM((DCPT, DCH), jnp.int32),
        pltpu.VMEM((DCH, HID), jnp.float32),
        pltpu.VMEM_SHARED((NP, HID), jnp.float32),
        pltpu.SemaphoreType.DMA((DDEPTH,)),
    ],
)
def _deg_sc(dst_hbm, ones_hbm, zz_hbm, out_hbm, dall, ones_v, dacc, sS):
    c = lax.axis_index("c")
    s = lax.axis_index("s")
    wid = s * NC + c
    pltpu.sync_copy(zz_hbm.at[pl.ds(s * RPS, RPS)], dacc.at[pl.ds(s * RPS, RPS)])
    pltpu.sync_copy(dst_hbm.at[pl.ds(wid * DCPT, DCPT)], dall)
    pltpu.sync_copy(ones_hbm, ones_v)
    plsc.subcore_barrier()

    def body(j, carry):
        q = lax.rem(j, DDEPTH)

        @pl.when(j >= DDEPTH)
        def _():
            pltpu.make_async_copy(ones_v, dacc.at[dall.at[j - DDEPTH]],
                                  sS.at[q]).wait()

        pltpu.async_copy(ones_v, dacc.at[dall.at[j]], sS.at[q], add=True)
        return carry

    lax.fori_loop(0, DCPT, body, 0)
    for k in range(DCPT - DDEPTH, DCPT):
        pltpu.make_async_copy(ones_v, dacc.at[dall.at[k]],
                              sS.at[k % DDEPTH]).wait()
    plsc.subcore_barrier()
    pltpu.sync_copy(dacc.at[pl.ds(s * RPS, RPS)],
                    out_hbm.at[c, pl.ds(s * RPS, RPS)])


@functools.partial(
    pl.kernel,
    out_type=jax.ShapeDtypeStruct((NC, NP, HID), jnp.float32),
    mesh=_mesh,
    scratch_types=[
        pltpu.VMEM((IRING, 2, CH), jnp.int32),
        pltpu.VMEM((DEPTH, CH, HID), jnp.float32),
        pltpu.VMEM_SHARED((TROWS, HID), jnp.float32),
        pltpu.VMEM_SHARED((HR, HID), jnp.float32),
        pltpu.SemaphoreType.DMA((IRING,)),
        pltpu.SemaphoreType.DMA((DEPTH,)),
        pltpu.SemaphoreType.DMA((DEPTH,)),
    ],
)
def _agg_sc(g_hbm, e0_hbm, e1_hbm, zz_hbm, out_hbm,
            ring, rbuf, table, acc, sI, sG, sS):
    c = lax.axis_index("c")
    s = lax.axis_index("s")
    wid = s * NC + c

    @pl.when(s < NS - 1)
    def _():
        pltpu.sync_copy(g_hbm.at[pl.ds(s * TRS, TRS)],
                        table.at[pl.ds(s * TRS, TRS)])

    @pl.when(s == NS - 1)
    def _():
        pltpu.sync_copy(g_hbm.at[pl.ds((NS - 1) * TRS, TLAST)],
                        table.at[pl.ds((NS - 1) * TRS, TLAST)])

    @pl.when(s == 0)
    def _():
        pltpu.sync_copy(zz_hbm.at[pl.ds(0, 1)],
                        table.at[pl.ds(ZROW, 1)])

    for t in range(2):
        eidx_hbm = e0_hbm if t == 0 else e1_hbm
        pltpu.sync_copy(g_hbm.at[pl.ds(t * HR + s * HRS, HRS)],
                        acc.at[pl.ds(s * HRS, HRS)])
        plsc.subcore_barrier()

        for k in range(IP):
            pltpu.async_copy(eidx_hbm.at[wid * CPT + k], ring.at[k], sI.at[k])
        for k in range(DEPTH - 1):
            pltpu.make_async_copy(eidx_hbm.at[wid * CPT + k], ring.at[k],
                                  sI.at[k]).wait()
            pltpu.async_copy(table.at[ring.at[k, 0]], rbuf.at[k], sG.at[k])

        def body(j, carry):
            q = lax.rem(j, DEPTH)
            r = lax.rem(j, IRING)
            pltpu.make_async_copy(table.at[ring.at[r, 0]], rbuf.at[q],
                                  sG.at[q]).wait()
            pltpu.async_copy(rbuf.at[q], acc.at[ring.at[r, 1]], sS.at[q],
                             add=True)

            @pl.when(j >= 1)
            def _():
                qs = lax.rem(j + DEPTH - 1, DEPTH)
                rs = lax.rem(j + IRING - 1, IRING)
                pltpu.make_async_copy(rbuf.at[qs], acc.at[ring.at[rs, 1]],
                                      sS.at[qs]).wait()

            @pl.when(j + DEPTH - 1 < CPT)
            def _():
                qn = lax.rem(j + DEPTH - 1, DEPTH)
                rn = lax.rem(j + DEPTH - 1, IRING)
                pltpu.make_async_copy(eidx_hbm.at[wid * CPT + j + DEPTH - 1],
                                      ring.at[rn], sI.at[rn]).wait()
                pltpu.async_copy(table.at[ring.at[rn, 0]], rbuf.at[qn],
                                 sG.at[qn])

            @pl.when(j + IP < CPT)
            def _():
                rp = lax.rem(j + IP, IRING)
                pltpu.async_copy(eidx_hbm.at[wid * CPT + j + IP],
                                 ring.at[rp], sI.at[rp])

            return carry

        lax.fori_loop(0, CPT, body, 0)
        pltpu.make_async_copy(rbuf.at[(CPT - 1) % DEPTH],
                              acc.at[ring.at[(CPT - 1) % IRING, 1]],
                              sS.at[(CPT - 1) % DEPTH]).wait()
        plsc.subcore_barrier()
        pltpu.sync_copy(acc.at[pl.ds(s * HRS, HRS)],
                        out_hbm.at[c, pl.ds(t * HR + s * HRS, HRS)])
        plsc.subcore_barrier()


def _erf(z):
    return lax.erf(z)


def _mm_t(a, b):
    return lax.dot_general(a, b, (((1,), (1,)), ((), ())),
                           preferred_element_type=jnp.float32)


def _tc_first_body(x_ref, w0_ref, b0_ref, wc_ref, deg_ref, g_ref, dinv_ref):
    x = x_ref[...]
    h = _erf(_C * (_mm_t(x, w0_ref[...]) + b0_ref[...]))
    deg = deg_ref[0, :, 0:1] + deg_ref[1, :, 0:1] + 1.0
    dinv = lax.rsqrt(deg)
    dinv_ref[...] = dinv
    g_ref[...] = dinv * _mm_t(h, wc_ref[...])


_tc_first = pl.pallas_call(
    _tc_first_body,
    grid=(GRID,),
    in_specs=[
        pl.BlockSpec((BLK, IN_CH), lambda i: (i, 0)),
        pl.BlockSpec((HID, IN_CH), lambda i: (0, 0)),
        pl.BlockSpec((HID,), lambda i: (0,)),
        pl.BlockSpec((HID, HID), lambda i: (0, 0)),
        pl.BlockSpec((NC, BLK, HID), lambda i: (0, i, 0)),
    ],
    out_specs=[
        pl.BlockSpec((BLK, HID), lambda i: (i, 0)),
        pl.BlockSpec((BLK, 1), lambda i: (i, 0)),
    ],
    out_shape=[
        jax.ShapeDtypeStruct((NP, HID), jnp.float32),
        jax.ShapeDtypeStruct((NP, 1), jnp.float32),
    ],
)


def _tc_mid_body(acc_ref, g_ref, dinv_ref, b_ref, w_ref, out_ref):
    dinv = dinv_ref[...]
    a = acc_ref[0] + acc_ref[1] - g_ref[...]
    h = _erf(_C * (dinv * a + b_ref[...]))
    out_ref[...] = dinv * _mm_t(h, w_ref[...])


_tc_mid = pl.pallas_call(
    _tc_mid_body,
    grid=(GRID,),
    in_specs=[
        pl.BlockSpec((NC, BLK, HID), lambda i: (0, i, 0)),
        pl.BlockSpec((BLK, HID), lambda i: (i, 0)),
        pl.BlockSpec((BLK, 1), lambda i: (i, 0)),
        pl.BlockSpec((HID,), lambda i: (0,)),
        pl.BlockSpec((HID, HID), lambda i: (0, 0)),
    ],
    out_specs=pl.BlockSpec((BLK, HID), lambda i: (i, 0)),
    out_shape=jax.ShapeDtypeStruct((NP, HID), jnp.float32),
)


def _tc_last_body(acc_ref, g_ref, dinv_ref, b_ref, wl_ref, bl_ref, out_ref):
    dinv = dinv_ref[...]
    a = acc_ref[0] + acc_ref[1] - g_ref[...]
    h = _erf(_C * (dinv * a + b_ref[...]))
    out_ref[...] = _mm_t(h, wl_ref[...]) + bl_ref[...]


_tc_last = pl.pallas_call(
    _tc_last_body,
    grid=(GRID,),
    in_specs=[
        pl.BlockSpec((NC, BLK, HID), lambda i: (0, i, 0)),
        pl.BlockSpec((BLK, HID), lambda i: (i, 0)),
        pl.BlockSpec((BLK, 1), lambda i: (i, 0)),
        pl.BlockSpec((HID,), lambda i: (0,)),
        pl.BlockSpec((OUT_CH, HID), lambda i: (0, 0)),
        pl.BlockSpec((OUT_CH,), lambda i: (0,)),
    ],
    out_specs=pl.BlockSpec((BLK, OUT_CH), lambda i: (i, 0)),
    out_shape=jax.ShapeDtypeStruct((NP, OUT_CH), jnp.float32),
)


def kernel(x, edge_index, W0, b0, Wc, bc, Wl, bl):
    xp = jnp.concatenate(
        [x, jnp.zeros((NP - N, IN_CH), jnp.float32)], axis=0)
    src = edge_index[0].reshape(NW, E // NW)
    dst = edge_index[1].reshape(NW, E // NW)
    pad_src = jnp.zeros((NW, PAD), jnp.int32)
    pad_dst = jnp.broadcast_to(N + jnp.arange(PAD, dtype=jnp.int32), (NW, PAD))
    srcp = jnp.concatenate([src, pad_src], axis=1).reshape(-1)
    dstp = jnp.concatenate([dst, pad_dst], axis=1).reshape(-1)

    in0 = dstp < HR
    s0 = jnp.where(in0, srcp, ZROW).reshape(NW * CPT, CH)
    s1 = jnp.where(in0, ZROW, srcp).reshape(NW * CPT, CH)
    dm = jnp.where(in0, dstp, dstp - HR).reshape(NW * CPT, CH)
    e0 = jnp.stack([s0, dm], axis=1)
    e1 = jnp.stack([s1, dm], axis=1)

    dst_deg = dstp.reshape(NW * DCPT, DCH)
    ones_rows = jnp.ones((DCH, HID), jnp.float32)
    zz = jnp.zeros((NP, HID), jnp.float32)

    degpart = _deg_sc(dst_deg, ones_rows, zz)
    g, dinv = _tc_first(xp, W0, b0, Wc[0], degpart)
    for layer in range(1, NUM_LAYERS):
        acc = _agg_sc(g, e0, e1, zz)
        g = _tc_mid(acc, g, dinv, bc[layer - 1], Wc[layer])
    acc = _agg_sc(g, e0, e1, zz)
    logits = _tc_last(acc, g, dinv, bc[NUM_LAYERS - 1], Wl, bl)
    return logits[:N]


# speedup vs baseline: 1.0728x; 1.0728x over previous
import functools
import math

import jax
import jax.numpy as jnp
from jax import lax
from jax.experimental import pallas as pl
from jax.experimental.pallas import tpu as pltpu
from jax.experimental.pallas import tpu_sc as plsc

N = 10000
E = 320000
IN_CH = 128
HID = 128
OUT_CH = 40
NUM_LAYERS = 4

NC = 2
NS = 16
NW = NC * NS
NP = 10240
RPS = NP // NS
CH = 16
EPT = NP
CPT = EPT // CH
PAD = EPT - E // NW
DCH = 64
DCPT = EPT // DCH
HR = NP // 2
HRS = HR // NS
ZROW = 10000
TROWS = 10008
TRS = 632
TLAST = N - (NS - 1) * TRS

BLK = 512
GRID = NP // BLK

_C = math.sqrt(math.pi) / 2.0

_mesh = plsc.VectorSubcoreMesh(core_axis_name="c", subcore_axis_name="s")


DEPTH = 3
IRING = 8
IP = 6
DDEPTH = 4


@functools.partial(
    pl.kernel,
    out_type=jax.ShapeDtypeStruct((NC, NP, HID), jnp.float32),
    mesh=_mesh,
    scratch_types=[
        pltpu.VMEM((DCPT, DCH), jnp.int32),
        pltpu.VMEM((DCH, HID), jnp.float32),
        pltpu.VMEM_SHARED((NP, HID), jnp.float32),
        pltpu.SemaphoreType.DMA((DDEPTH,)),
    ],
)
def _deg_sc(dst_hbm, ones_hbm, zz_hbm, out_hbm, dall, ones_v, dacc, sS):
    c = lax.axis_index("c")
    s = lax.axis_index("s")
    wid = s * NC + c
    pltpu.sync_copy(zz_hbm.at[pl.ds(s * RPS, RPS)], dacc.at[pl.ds(s * RPS, RPS)])
    pltpu.sync_copy(dst_hbm.at[pl.ds(wid * DCPT, DCPT)], dall)
    pltpu.sync_copy(ones_hbm, ones_v)
    plsc.subcore_barrier()

    def body(j, carry):
        q = lax.rem(j, DDEPTH)

        @pl.when(j >= DDEPTH)
        def _():
            pltpu.make_async_copy(ones_v, dacc.at[dall.at[j - DDEPTH]],
                                  sS.at[q]).wait()

        pltpu.async_copy(ones_v, dacc.at[dall.at[j]], sS.at[q], add=True)
        return carry

    lax.fori_loop(0, DCPT, body, 0)
    for k in range(DCPT - DDEPTH, DCPT):
        pltpu.make_async_copy(ones_v, dacc.at[dall.at[k]],
                              sS.at[k % DDEPTH]).wait()
    plsc.subcore_barrier()
    pltpu.sync_copy(dacc.at[pl.ds(s * RPS, RPS)],
                    out_hbm.at[c, pl.ds(s * RPS, RPS)])


@functools.partial(
    pl.kernel,
    out_type=jax.ShapeDtypeStruct((NC, NP, HID), jnp.float32),
    mesh=_mesh,
    scratch_types=[
        pltpu.VMEM((IRING, 2, CH), jnp.int32),
        pltpu.VMEM((DEPTH, CH, HID), jnp.float32),
        pltpu.VMEM_SHARED((TROWS, HID), jnp.float32),
        pltpu.VMEM_SHARED((HR, HID), jnp.float32),
        pltpu.SemaphoreType.DMA((IRING,)),
        pltpu.SemaphoreType.DMA((DEPTH,)),
        pltpu.SemaphoreType.DMA((DEPTH,)),
    ],
)
def _agg_sc(g_hbm, e0_hbm, e1_hbm, zz_hbm, out_hbm,
            ring, rbuf, table, acc, sI, sG, sS):
    c = lax.axis_index("c")
    s = lax.axis_index("s")
    wid = s * NC + c

    @pl.when(s < NS - 1)
    def _():
        pltpu.sync_copy(g_hbm.at[pl.ds(s * TRS, TRS)],
                        table.at[pl.ds(s * TRS, TRS)])

    @pl.when(s == NS - 1)
    def _():
        pltpu.sync_copy(g_hbm.at[pl.ds((NS - 1) * TRS, TLAST)],
                        table.at[pl.ds((NS - 1) * TRS, TLAST)])

    @pl.when(s == 0)
    def _():
        pltpu.sync_copy(zz_hbm.at[pl.ds(0, TROWS - ZROW)],
                        table.at[pl.ds(ZROW, TROWS - ZROW)])

    for t in range(2):
        eidx_hbm = e0_hbm if t == 0 else e1_hbm
        pltpu.sync_copy(g_hbm.at[pl.ds(t * HR + s * HRS, HRS)],
                        acc.at[pl.ds(s * HRS, HRS)])
        plsc.subcore_barrier()

        for k in range(IP):
            pltpu.async_copy(eidx_hbm.at[wid * CPT + k], ring.at[k], sI.at[k])
        for k in range(DEPTH - 1):
            pltpu.make_async_copy(eidx_hbm.at[wid * CPT + k], ring.at[k],
                                  sI.at[k]).wait()
            pltpu.async_copy(table.at[ring.at[k, 0]], rbuf.at[k], sG.at[k])

        def body(j, carry):
            q = lax.rem(j, DEPTH)
            r = lax.rem(j, IRING)
            pltpu.make_async_copy(table.at[ring.at[r, 0]], rbuf.at[q],
                                  sG.at[q]).wait()
            pltpu.async_copy(rbuf.at[q], acc.at[ring.at[r, 1]], sS.at[q],
                             add=True)

            @pl.when(j >= 1)
            def _():
                qs = lax.rem(j + DEPTH - 1, DEPTH)
                rs = lax.rem(j + IRING - 1, IRING)
                pltpu.make_async_copy(rbuf.at[qs], acc.at[ring.at[rs, 1]],
                                      sS.at[qs]).wait()

            @pl.when(j + DEPTH - 1 < CPT)
            def _():
                qn = lax.rem(j + DEPTH - 1, DEPTH)
                rn = lax.rem(j + DEPTH - 1, IRING)
                pltpu.make_async_copy(eidx_hbm.at[wid * CPT + j + DEPTH - 1],
                                      ring.at[rn], sI.at[rn]).wait()
                pltpu.async_copy(table.at[ring.at[rn, 0]], rbuf.at[qn],
                                 sG.at[qn])

            @pl.when(j + IP < CPT)
            def _():
                rp = lax.rem(j + IP, IRING)
                pltpu.async_copy(eidx_hbm.at[wid * CPT + j + IP],
                                 ring.at[rp], sI.at[rp])

            return carry

        lax.fori_loop(0, CPT, body, 0)
        pltpu.make_async_copy(rbuf.at[(CPT - 1) % DEPTH],
                              acc.at[ring.at[(CPT - 1) % IRING, 1]],
                              sS.at[(CPT - 1) % DEPTH]).wait()
        plsc.subcore_barrier()
        pltpu.sync_copy(acc.at[pl.ds(s * HRS, HRS)],
                        out_hbm.at[c, pl.ds(t * HR + s * HRS, HRS)])
        plsc.subcore_barrier()


def _erf(z):
    return lax.erf(z)


def _mm_t(a, b):
    return lax.dot_general(a, b, (((1,), (1,)), ((), ())),
                           preferred_element_type=jnp.float32)


def _tc_first_body(x_ref, w0_ref, b0_ref, wc_ref, deg_ref, g_ref, dinv_ref):
    x = x_ref[...]
    h = _erf(_C * (_mm_t(x, w0_ref[...]) + b0_ref[...]))
    deg = deg_ref[0, :, 0:1] + deg_ref[1, :, 0:1] + 1.0
    dinv = lax.rsqrt(deg)
    dinv_ref[...] = dinv
    g_ref[...] = dinv * _mm_t(h, wc_ref[...])


_tc_first = pl.pallas_call(
    _tc_first_body,
    grid=(GRID,),
    in_specs=[
        pl.BlockSpec((BLK, IN_CH), lambda i: (i, 0)),
        pl.BlockSpec((HID, IN_CH), lambda i: (0, 0)),
        pl.BlockSpec((HID,), lambda i: (0,)),
        pl.BlockSpec((HID, HID), lambda i: (0, 0)),
        pl.BlockSpec((NC, BLK, HID), lambda i: (0, i, 0)),
    ],
    out_specs=[
        pl.BlockSpec((BLK, HID), lambda i: (i, 0)),
        pl.BlockSpec((BLK, 1), lambda i: (i, 0)),
    ],
    out_shape=[
        jax.ShapeDtypeStruct((NP, HID), jnp.float32),
        jax.ShapeDtypeStruct((NP, 1), jnp.float32),
    ],
)


def _tc_mid_body(acc_ref, g_ref, dinv_ref, b_ref, w_ref, out_ref):
    dinv = dinv_ref[...]
    a = acc_ref[0] + acc_ref[1] - g_ref[...]
    h = _erf(_C * (dinv * a + b_ref[...]))
    out_ref[...] = dinv * _mm_t(h, w_ref[...])


_tc_mid = pl.pallas_call(
    _tc_mid_body,
    grid=(GRID,),
    in_specs=[
        pl.BlockSpec((NC, BLK, HID), lambda i: (0, i, 0)),
        pl.BlockSpec((BLK, HID), lambda i: (i, 0)),
        pl.BlockSpec((BLK, 1), lambda i: (i, 0)),
        pl.BlockSpec((HID,), lambda i: (0,)),
        pl.BlockSpec((HID, HID), lambda i: (0, 0)),
    ],
    out_specs=pl.BlockSpec((BLK, HID), lambda i: (i, 0)),
    out_shape=jax.ShapeDtypeStruct((NP, HID), jnp.float32),
)


def _tc_last_body(acc_ref, g_ref, dinv_ref, b_ref, wl_ref, bl_ref, out_ref):
    dinv = dinv_ref[...]
    a = acc_ref[0] + acc_ref[1] - g_ref[...]
    h = _erf(_C * (dinv * a + b_ref[...]))
    out_ref[...] = _mm_t(h, wl_ref[...]) + bl_ref[...]


_tc_last = pl.pallas_call(
    _tc_last_body,
    grid=(GRID,),
    in_specs=[
        pl.BlockSpec((NC, BLK, HID), lambda i: (0, i, 0)),
        pl.BlockSpec((BLK, HID), lambda i: (i, 0)),
        pl.BlockSpec((BLK, 1), lambda i: (i, 0)),
        pl.BlockSpec((HID,), lambda i: (0,)),
        pl.BlockSpec((OUT_CH, HID), lambda i: (0, 0)),
        pl.BlockSpec((OUT_CH,), lambda i: (0,)),
    ],
    out_specs=pl.BlockSpec((BLK, OUT_CH), lambda i: (i, 0)),
    out_shape=jax.ShapeDtypeStruct((NP, OUT_CH), jnp.float32),
)


def kernel(x, edge_index, W0, b0, Wc, bc, Wl, bl):
    xp = jnp.concatenate(
        [x, jnp.zeros((NP - N, IN_CH), jnp.float32)], axis=0)
    src = edge_index[0].reshape(NW, E // NW)
    dst = edge_index[1].reshape(NW, E // NW)
    pad_src = jnp.zeros((NW, PAD), jnp.int32)
    pad_dst = jnp.broadcast_to(N + jnp.arange(PAD, dtype=jnp.int32), (NW, PAD))
    srcp = jnp.concatenate([src, pad_src], axis=1).reshape(-1)
    dstp = jnp.concatenate([dst, pad_dst], axis=1).reshape(-1)

    in0 = dstp < HR
    s0 = jnp.where(in0, srcp, ZROW).reshape(NW * CPT, CH)
    s1 = jnp.where(in0, ZROW, srcp).reshape(NW * CPT, CH)
    dm = jnp.where(in0, dstp, dstp - HR).reshape(NW * CPT, CH)
    e0 = jnp.stack([s0, dm], axis=1)
    e1 = jnp.stack([s1, dm], axis=1)

    dst_deg = dstp.reshape(NW * DCPT, DCH)
    ones_rows = jnp.ones((DCH, HID), jnp.float32)
    zz = jnp.zeros((NP, HID), jnp.float32)

    degpart = _deg_sc(dst_deg, ones_rows, zz)
    g, dinv = _tc_first(xp, W0, b0, Wc[0], degpart)
    for layer in range(1, NUM_LAYERS):
        acc = _agg_sc(g, e0, e1, zz)
        g = _tc_mid(acc, g, dinv, bc[layer - 1], Wc[layer])
    acc = _agg_sc(g, e0, e1, zz)
    logits = _tc_last(acc, g, dinv, bc[NUM_LAYERS - 1], Wl, bl)
    return logits[:N]


# speedup vs baseline: 1.0963x; 1.0218x over previous
import functools
import math

import jax
import jax.numpy as jnp
from jax import lax
from jax.experimental import pallas as pl
from jax.experimental.pallas import tpu as pltpu
from jax.experimental.pallas import tpu_sc as plsc

N = 10000
E = 320000
IN_CH = 128
HID = 128
OUT_CH = 40
NUM_LAYERS = 4

NC = 2
NS = 16
NW = NC * NS
NP = 10240
RPS = NP // NS
CH = 16
EPT = NP
CPT = EPT // CH
PAD = EPT - E // NW
DCH = 64
DCPT = EPT // DCH
HR = NP // 2
HRS = HR // NS
ZROW = 10000
TROWS = 10008
TRS = 632
TLAST = N - (NS - 1) * TRS

BLK = 512
GRID = NP // BLK

_C = math.sqrt(math.pi) / 2.0

_mesh = plsc.VectorSubcoreMesh(core_axis_name="c", subcore_axis_name="s")


DEPTH = 3
IRING = 8
IP = 6
DDEPTH = 4


@functools.partial(
    pl.kernel,
    out_type=jax.ShapeDtypeStruct((NC, NP, HID), jnp.float32),
    mesh=_mesh,
    scratch_types=[
        pltpu.VMEM((DCPT, DCH), jnp.int32),
        pltpu.VMEM((DCH, HID), jnp.float32),
        pltpu.VMEM_SHARED((NP, HID), jnp.float32),
        pltpu.SemaphoreType.DMA((DDEPTH,)),
    ],
)
def _deg_sc(dst_hbm, ones_hbm, zz_hbm, out_hbm, dall, ones_v, dacc, sS):
    c = lax.axis_index("c")
    s = lax.axis_index("s")
    wid = s * NC + c
    pltpu.sync_copy(zz_hbm.at[pl.ds(s * RPS, RPS)], dacc.at[pl.ds(s * RPS, RPS)])
    pltpu.sync_copy(dst_hbm.at[pl.ds(wid * DCPT, DCPT)], dall)
    pltpu.sync_copy(ones_hbm, ones_v)
    plsc.subcore_barrier()

    def body(j, carry):
        q = lax.rem(j, DDEPTH)

        @pl.when(j >= DDEPTH)
        def _():
            pltpu.make_async_copy(ones_v, dacc.at[dall.at[j - DDEPTH]],
                                  sS.at[q]).wait()

        pltpu.async_copy(ones_v, dacc.at[dall.at[j]], sS.at[q], add=True)
        return carry

    lax.fori_loop(0, DCPT, body, 0)
    for k in range(DCPT - DDEPTH, DCPT):
        pltpu.make_async_copy(ones_v, dacc.at[dall.at[k]],
                              sS.at[k % DDEPTH]).wait()
    plsc.subcore_barrier()
    pltpu.sync_copy(dacc.at[pl.ds(s * RPS, RPS)],
                    out_hbm.at[c, pl.ds(s * RPS, RPS)])


@functools.partial(
    pl.kernel,
    out_type=jax.ShapeDtypeStruct((NC, NP, HID), jnp.float32),
    mesh=_mesh,
    scratch_types=[
        pltpu.VMEM((IRING, 1, CH), jnp.int32),
        pltpu.VMEM((IRING, 1, CH), jnp.int32),
        pltpu.VMEM((DEPTH, CH, HID), jnp.float32),
        pltpu.VMEM_SHARED((TROWS, HID), jnp.float32),
        pltpu.VMEM_SHARED((HR, HID), jnp.float32),
        pltpu.SemaphoreType.DMA((IRING,)),
        pltpu.SemaphoreType.DMA((IRING,)),
        pltpu.SemaphoreType.DMA((DEPTH,)),
        pltpu.SemaphoreType.DMA((DEPTH,)),
    ],
)
def _agg_sc(g_hbm, s0_hbm, s1_hbm, dm_hbm, zz_hbm, out_hbm,
            sring, dring, rbuf, table, acc, sI, sJ, sG, sS):
    c = lax.axis_index("c")
    s = lax.axis_index("s")
    wid = s * NC + c

    @pl.when(s < NS - 1)
    def _():
        pltpu.sync_copy(g_hbm.at[pl.ds(s * TRS, TRS)],
                        table.at[pl.ds(s * TRS, TRS)])

    @pl.when(s == NS - 1)
    def _():
        pltpu.sync_copy(g_hbm.at[pl.ds((NS - 1) * TRS, TLAST)],
                        table.at[pl.ds((NS - 1) * TRS, TLAST)])

    @pl.when(s == 0)
    def _():
        pltpu.sync_copy(zz_hbm.at[pl.ds(0, TROWS - ZROW)],
                        table.at[pl.ds(ZROW, TROWS - ZROW)])

    for t in range(2):
        sidx_hbm = s0_hbm if t == 0 else s1_hbm
        pltpu.sync_copy(g_hbm.at[pl.ds(t * HR + s * HRS, HRS)],
                        acc.at[pl.ds(s * HRS, HRS)])
        plsc.subcore_barrier()

        for k in range(IP):
            pltpu.async_copy(sidx_hbm.at[wid * CPT + k], sring.at[k], sI.at[k])
            pltpu.async_copy(dm_hbm.at[wid * CPT + k], dring.at[k], sJ.at[k])
        for k in range(DEPTH - 1):
            pltpu.make_async_copy(sidx_hbm.at[wid * CPT + k], sring.at[k],
                                  sI.at[k]).wait()
            pltpu.async_copy(table.at[sring.at[k, 0]], rbuf.at[k], sG.at[k])

        def body(j, carry):
            q = lax.rem(j, DEPTH)
            r = lax.rem(j, IRING)
            pltpu.make_async_copy(table.at[sring.at[r, 0]], rbuf.at[q],
                                  sG.at[q]).wait()
            pltpu.make_async_copy(dm_hbm.at[wid * CPT + j], dring.at[r],
                                  sJ.at[r]).wait()
            pltpu.async_copy(rbuf.at[q], acc.at[dring.at[r, 0]], sS.at[q],
                             add=True)

            @pl.when(j >= 1)
            def _():
                qs = lax.rem(j + DEPTH - 1, DEPTH)
                rs = lax.rem(j + IRING - 1, IRING)
                pltpu.make_async_copy(rbuf.at[qs], acc.at[dring.at[rs, 0]],
                                      sS.at[qs]).wait()

            @pl.when(j + DEPTH - 1 < CPT)
            def _():
                qn = lax.rem(j + DEPTH - 1, DEPTH)
                rn = lax.rem(j + DEPTH - 1, IRING)
                pltpu.make_async_copy(sidx_hbm.at[wid * CPT + j + DEPTH - 1],
                                      sring.at[rn], sI.at[rn]).wait()
                pltpu.async_copy(table.at[sring.at[rn, 0]], rbuf.at[qn],
                                 sG.at[qn])

            @pl.when(j + IP < CPT)
            def _():
                rp = lax.rem(j + IP, IRING)
                pltpu.async_copy(sidx_hbm.at[wid * CPT + j + IP],
                                 sring.at[rp], sI.at[rp])
                pltpu.async_copy(dm_hbm.at[wid * CPT + j + IP],
                                 dring.at[rp], sJ.at[rp])

            return carry

        lax.fori_loop(0, CPT, body, 0)
        pltpu.make_async_copy(rbuf.at[(CPT - 1) % DEPTH],
                              acc.at[dring.at[(CPT - 1) % IRING, 0]],
                              sS.at[(CPT - 1) % DEPTH]).wait()
        plsc.subcore_barrier()
        pltpu.sync_copy(acc.at[pl.ds(s * HRS, HRS)],
                        out_hbm.at[c, pl.ds(t * HR + s * HRS, HRS)])
        plsc.subcore_barrier()


def _erf(z):
    return lax.erf(z)


def _mm_t(a, b):
    return lax.dot_general(a, b, (((1,), (1,)), ((), ())),
                           preferred_element_type=jnp.float32)


def _tc_first_body(x_ref, w0_ref, b0_ref, wc_ref, deg_ref, g_ref, dinv_ref):
    x = x_ref[...]
    h = _erf(_C * (_mm_t(x, w0_ref[...]) + b0_ref[...]))
    deg = deg_ref[0, :, 0:1] + deg_ref[1, :, 0:1] + 1.0
    dinv = lax.rsqrt(deg)
    dinv_ref[...] = dinv
    g_ref[...] = dinv * _mm_t(h, wc_ref[...])


_tc_first = pl.pallas_call(
    _tc_first_body,
    grid=(GRID,),
    in_specs=[
        pl.BlockSpec((BLK, IN_CH), lambda i: (i, 0)),
        pl.BlockSpec((HID, IN_CH), lambda i: (0, 0)),
        pl.BlockSpec((HID,), lambda i: (0,)),
        pl.BlockSpec((HID, HID), lambda i: (0, 0)),
        pl.BlockSpec((NC, BLK, HID), lambda i: (0, i, 0)),
    ],
    out_specs=[
        pl.BlockSpec((BLK, HID), lambda i: (i, 0)),
        pl.BlockSpec((BLK, 1), lambda i: (i, 0)),
    ],
    out_shape=[
        jax.ShapeDtypeStruct((NP, HID), jnp.float32),
        jax.ShapeDtypeStruct((NP, 1), jnp.float32),
    ],
)


def _tc_mid_body(acc_ref, g_ref, dinv_ref, b_ref, w_ref, out_ref):
    dinv = dinv_ref[...]
    a = acc_ref[0] + acc_ref[1] - g_ref[...]
    h = _erf(_C * (dinv * a + b_ref[...]))
    out_ref[...] = dinv * _mm_t(h, w_ref[...])


_tc_mid = pl.pallas_call(
    _tc_mid_body,
    grid=(GRID,),
    in_specs=[
        pl.BlockSpec((NC, BLK, HID), lambda i: (0, i, 0)),
        pl.BlockSpec((BLK, HID), lambda i: (i, 0)),
        pl.BlockSpec((BLK, 1), lambda i: (i, 0)),
        pl.BlockSpec((HID,), lambda i: (0,)),
        pl.BlockSpec((HID, HID), lambda i: (0, 0)),
    ],
    out_specs=pl.BlockSpec((BLK, HID), lambda i: (i, 0)),
    out_shape=jax.ShapeDtypeStruct((NP, HID), jnp.float32),
)


def _tc_last_body(acc_ref, g_ref, dinv_ref, b_ref, wl_ref, bl_ref, out_ref):
    dinv = dinv_ref[...]
    a = acc_ref[0] + acc_ref[1] - g_ref[...]
    h = _erf(_C * (dinv * a + b_ref[...]))
    out_ref[...] = _mm_t(h, wl_ref[...]) + bl_ref[...]


_tc_last = pl.pallas_call(
    _tc_last_body,
    grid=(GRID,),
    in_specs=[
        pl.BlockSpec((NC, BLK, HID), lambda i: (0, i, 0)),
        pl.BlockSpec((BLK, HID), lambda i: (i, 0)),
        pl.BlockSpec((BLK, 1), lambda i: (i, 0)),
        pl.BlockSpec((HID,), lambda i: (0,)),
        pl.BlockSpec((OUT_CH, HID), lambda i: (0, 0)),
        pl.BlockSpec((OUT_CH,), lambda i: (0,)),
    ],
    out_specs=pl.BlockSpec((BLK, OUT_CH), lambda i: (i, 0)),
    out_shape=jax.ShapeDtypeStruct((NP, OUT_CH), jnp.float32),
)


def kernel(x, edge_index, W0, b0, Wc, bc, Wl, bl):
    xp = jnp.concatenate(
        [x, jnp.zeros((NP - N, IN_CH), jnp.float32)], axis=0)
    src = edge_index[0].reshape(NW, E // NW)
    dst = edge_index[1].reshape(NW, E // NW)
    pad_src = jnp.zeros((NW, PAD), jnp.int32)
    pad_dst = jnp.broadcast_to(N + jnp.arange(PAD, dtype=jnp.int32), (NW, PAD))
    srcp = jnp.concatenate([src, pad_src], axis=1).reshape(-1)
    dstp = jnp.concatenate([dst, pad_dst], axis=1).reshape(-1)

    in0 = dstp < HR
    s0 = jnp.where(in0, srcp, ZROW).reshape(NW * CPT, 1, CH)
    s1 = jnp.where(in0, ZROW, srcp).reshape(NW * CPT, 1, CH)
    dm = jnp.where(in0, dstp, dstp - HR).reshape(NW * CPT, 1, CH)

    dst_deg = dstp.reshape(NW * DCPT, DCH)
    ones_rows = jnp.ones((DCH, HID), jnp.float32)
    zz = jnp.zeros((NP, HID), jnp.float32)

    degpart = _deg_sc(dst_deg, ones_rows, zz)
    g, dinv = _tc_first(xp, W0, b0, Wc[0], degpart)
    for layer in range(1, NUM_LAYERS):
        acc = _agg_sc(g, s0, s1, dm, zz)
        g = _tc_mid(acc, g, dinv, bc[layer - 1], Wc[layer])
    acc = _agg_sc(g, s0, s1, dm, zz)
    logits = _tc_last(acc, g, dinv, bc[NUM_LAYERS - 1], Wl, bl)
    return logits[:N]


# speedup vs baseline: 1.1326x; 1.0331x over previous
import functools
import math

import jax
import jax.numpy as jnp
from jax import lax
from jax.experimental import pallas as pl
from jax.experimental.pallas import tpu as pltpu
from jax.experimental.pallas import tpu_sc as plsc

N = 10000
E = 320000
IN_CH = 128
HID = 128
OUT_CH = 40
NUM_LAYERS = 4

NC = 2
NS = 16
NW = NC * NS
NP = 10240
RPS = NP // NS
CH = 16
EPT = NP
CPT = EPT // CH
CPT2 = (NW * EPT) // (NS * CH)
PAD = EPT - E // NW
DCH = 64
DCPT = EPT // DCH
HR = NP // 2
HRS = HR // NS
ZROW = 10000
TROWS = 10008
TRS = 632
TLAST = N - (NS - 1) * TRS

BLK = 512
GRID = NP // BLK

_C = math.sqrt(math.pi) / 2.0

_mesh = plsc.VectorSubcoreMesh(core_axis_name="c", subcore_axis_name="s")


DEPTH = 3
IRING = 8
IP = 6
DDEPTH = 4


@functools.partial(
    pl.kernel,
    out_type=jax.ShapeDtypeStruct((NC, NP, HID), jnp.float32),
    mesh=_mesh,
    scratch_types=[
        pltpu.VMEM((DCPT, DCH), jnp.int32),
        pltpu.VMEM((DCH, HID), jnp.float32),
        pltpu.VMEM_SHARED((NP, HID), jnp.float32),
        pltpu.SemaphoreType.DMA((DDEPTH,)),
    ],
)
def _deg_sc(dst_hbm, ones_hbm, zz_hbm, out_hbm, dall, ones_v, dacc, sS):
    c = lax.axis_index("c")
    s = lax.axis_index("s")
    wid = s * NC + c
    pltpu.sync_copy(zz_hbm.at[pl.ds(s * RPS, RPS)], dacc.at[pl.ds(s * RPS, RPS)])
    pltpu.sync_copy(dst_hbm.at[pl.ds(wid * DCPT, DCPT)], dall)
    pltpu.sync_copy(ones_hbm, ones_v)
    plsc.subcore_barrier()

    def body(j, carry):
        q = lax.rem(j, DDEPTH)

        @pl.when(j >= DDEPTH)
        def _():
            pltpu.make_async_copy(ones_v, dacc.at[dall.at[j - DDEPTH]],
                                  sS.at[q]).wait()

        pltpu.async_copy(ones_v, dacc.at[dall.at[j]], sS.at[q], add=True)
        return carry

    lax.fori_loop(0, DCPT, body, 0)
    for k in range(DCPT - DDEPTH, DCPT):
        pltpu.make_async_copy(ones_v, dacc.at[dall.at[k]],
                              sS.at[k % DDEPTH]).wait()
    plsc.subcore_barrier()
    pltpu.sync_copy(dacc.at[pl.ds(s * RPS, RPS)],
                    out_hbm.at[c, pl.ds(s * RPS, RPS)])


@functools.partial(
    pl.kernel,
    out_type=jax.ShapeDtypeStruct((NC, HR, HID), jnp.float32),
    mesh=_mesh,
    scratch_types=[
        pltpu.VMEM((IRING, 1, CH), jnp.int32),
        pltpu.VMEM((IRING, 1, CH), jnp.int32),
        pltpu.VMEM((DEPTH, CH, HID), jnp.float32),
        pltpu.VMEM_SHARED((TROWS, HID), jnp.float32),
        pltpu.VMEM_SHARED((HR, HID), jnp.float32),
        pltpu.SemaphoreType.DMA((IRING,)),
        pltpu.SemaphoreType.DMA((IRING,)),
        pltpu.SemaphoreType.DMA((DEPTH,)),
        pltpu.SemaphoreType.DMA((DEPTH,)),
    ],
)
def _agg_sc(g_hbm, s_hbm, dm_hbm, zz_hbm, out_hbm,
            sring, dring, rbuf, table, acc, sI, sJ, sG, sS):
    c = lax.axis_index("c")
    s = lax.axis_index("s")

    @pl.when(s < NS - 1)
    def _():
        pltpu.sync_copy(g_hbm.at[pl.ds(s * TRS, TRS)],
                        table.at[pl.ds(s * TRS, TRS)])

    @pl.when(s == NS - 1)
    def _():
        pltpu.sync_copy(g_hbm.at[pl.ds((NS - 1) * TRS, TLAST)],
                        table.at[pl.ds((NS - 1) * TRS, TLAST)])

    @pl.when(s == 0)
    def _():
        pltpu.sync_copy(zz_hbm.at[pl.ds(0, TROWS - ZROW)],
                        table.at[pl.ds(ZROW, TROWS - ZROW)])

    pltpu.sync_copy(g_hbm.at[pl.ds(c * HR + s * HRS, HRS)],
                    acc.at[pl.ds(s * HRS, HRS)])
    plsc.subcore_barrier()

    for k in range(IP):
        pltpu.async_copy(s_hbm.at[c, s * CPT2 + k], sring.at[k], sI.at[k])
        pltpu.async_copy(dm_hbm.at[s * CPT2 + k], dring.at[k], sJ.at[k])
    for k in range(DEPTH - 1):
        pltpu.make_async_copy(s_hbm.at[c, s * CPT2 + k], sring.at[k],
                              sI.at[k]).wait()
        pltpu.async_copy(table.at[sring.at[k, 0]], rbuf.at[k], sG.at[k])

    def body(j, carry):
        q = lax.rem(j, DEPTH)
        r = lax.rem(j, IRING)
        pltpu.make_async_copy(table.at[sring.at[r, 0]], rbuf.at[q],
                              sG.at[q]).wait()
        pltpu.make_async_copy(dm_hbm.at[s * CPT2 + j], dring.at[r],
                              sJ.at[r]).wait()
        pltpu.async_copy(rbuf.at[q], acc.at[dring.at[r, 0]], sS.at[q],
                         add=True)

        @pl.when(j >= 1)
        def _():
            qs = lax.rem(j + DEPTH - 1, DEPTH)
            rs = lax.rem(j + IRING - 1, IRING)
            pltpu.make_async_copy(rbuf.at[qs], acc.at[dring.at[rs, 0]],
                                  sS.at[qs]).wait()

        @pl.when(j + DEPTH - 1 < CPT2)
        def _():
            qn = lax.rem(j + DEPTH - 1, DEPTH)
            rn = lax.rem(j + DEPTH - 1, IRING)
            pltpu.make_async_copy(s_hbm.at[c, s * CPT2 + j + DEPTH - 1],
                                  sring.at[rn], sI.at[rn]).wait()
            pltpu.async_copy(table.at[sring.at[rn, 0]], rbuf.at[qn],
                             sG.at[qn])

        @pl.when(j + IP < CPT2)
        def _():
            rp = lax.rem(j + IP, IRING)
            pltpu.async_copy(s_hbm.at[c, s * CPT2 + j + IP],
                             sring.at[rp], sI.at[rp])
            pltpu.async_copy(dm_hbm.at[s * CPT2 + j + IP],
                             dring.at[rp], sJ.at[rp])

        return carry

    lax.fori_loop(0, CPT2, body, 0)
    pltpu.make_async_copy(rbuf.at[(CPT2 - 1) % DEPTH],
                          acc.at[dring.at[(CPT2 - 1) % IRING, 0]],
                          sS.at[(CPT2 - 1) % DEPTH]).wait()
    plsc.subcore_barrier()
    pltpu.sync_copy(acc.at[pl.ds(s * HRS, HRS)],
                    out_hbm.at[c, pl.ds(s * HRS, HRS)])


def _erf(z):
    return lax.erf(z)


def _mm_t(a, b):
    return lax.dot_general(a, b, (((1,), (1,)), ((), ())),
                           preferred_element_type=jnp.float32)


def _tc_first_body(x_ref, w0_ref, b0_ref, wc_ref, deg_ref, g_ref, dinv_ref):
    x = x_ref[...]
    h = _erf(_C * (_mm_t(x, w0_ref[...]) + b0_ref[...]))
    deg = deg_ref[0, :, 0:1] + deg_ref[1, :, 0:1] + 1.0
    dinv = lax.rsqrt(deg)
    dinv_ref[...] = dinv
    g_ref[...] = dinv * _mm_t(h, wc_ref[...])


_tc_first = pl.pallas_call(
    _tc_first_body,
    grid=(GRID,),
    in_specs=[
        pl.BlockSpec((BLK, IN_CH), lambda i: (i, 0)),
        pl.BlockSpec((HID, IN_CH), lambda i: (0, 0)),
        pl.BlockSpec((HID,), lambda i: (0,)),
        pl.BlockSpec((HID, HID), lambda i: (0, 0)),
        pl.BlockSpec((NC, BLK, HID), lambda i: (0, i, 0)),
    ],
    out_specs=[
        pl.BlockSpec((BLK, HID), lambda i: (i, 0)),
        pl.BlockSpec((BLK, 1), lambda i: (i, 0)),
    ],
    out_shape=[
        jax.ShapeDtypeStruct((NP, HID), jnp.float32),
        jax.ShapeDtypeStruct((NP, 1), jnp.float32),
    ],
)


def _tc_mid_body(acc_ref, dinv_ref, b_ref, w_ref, out_ref):
    dinv = dinv_ref[...]
    h = _erf(_C * (dinv * acc_ref[...] + b_ref[...]))
    out_ref[...] = dinv * _mm_t(h, w_ref[...])


_tc_mid = pl.pallas_call(
    _tc_mid_body,
    grid=(GRID,),
    in_specs=[
        pl.BlockSpec((BLK, HID), lambda i: (i, 0)),
        pl.BlockSpec((BLK, 1), lambda i: (i, 0)),
        pl.BlockSpec((HID,), lambda i: (0,)),
        pl.BlockSpec((HID, HID), lambda i: (0, 0)),
    ],
    out_specs=pl.BlockSpec((BLK, HID), lambda i: (i, 0)),
    out_shape=jax.ShapeDtypeStruct((NP, HID), jnp.float32),
)


def _tc_last_body(acc_ref, dinv_ref, b_ref, wl_ref, bl_ref, out_ref):
    dinv = dinv_ref[...]
    h = _erf(_C * (dinv * acc_ref[...] + b_ref[...]))
    out_ref[...] = _mm_t(h, wl_ref[...]) + bl_ref[...]


_tc_last = pl.pallas_call(
    _tc_last_body,
    grid=(GRID,),
    in_specs=[
        pl.BlockSpec((BLK, HID), lambda i: (i, 0)),
        pl.BlockSpec((BLK, 1), lambda i: (i, 0)),
        pl.BlockSpec((HID,), lambda i: (0,)),
        pl.BlockSpec((OUT_CH, HID), lambda i: (0, 0)),
        pl.BlockSpec((OUT_CH,), lambda i: (0,)),
    ],
    out_specs=pl.BlockSpec((BLK, OUT_CH), lambda i: (i, 0)),
    out_shape=jax.ShapeDtypeStruct((NP, OUT_CH), jnp.float32),
)


def kernel(x, edge_index, W0, b0, Wc, bc, Wl, bl):
    xp = jnp.concatenate(
        [x, jnp.zeros((NP - N, IN_CH), jnp.float32)], axis=0)
    src = edge_index[0].reshape(NW, E // NW)
    dst = edge_index[1].reshape(NW, E // NW)
    pad_src = jnp.zeros((NW, PAD), jnp.int32)
    pad_dst = jnp.broadcast_to(N + jnp.arange(PAD, dtype=jnp.int32), (NW, PAD))
    srcp = jnp.concatenate([src, pad_src], axis=1).reshape(-1)
    dstp = jnp.concatenate([dst, pad_dst], axis=1).reshape(-1)

    in0 = dstp < HR
    s0 = jnp.where(in0, srcp, ZROW).reshape(NS * CPT2, 1, CH)
    s1 = jnp.where(in0, ZROW, srcp).reshape(NS * CPT2, 1, CH)
    s01 = jnp.stack([s0, s1])
    dm = jnp.where(in0, dstp, dstp - HR).reshape(NS * CPT2, 1, CH)

    dst_deg = dstp.reshape(NW * DCPT, DCH)
    ones_rows = jnp.ones((DCH, HID), jnp.float32)
    zz = jnp.zeros((NP, HID), jnp.float32)

    degpart = _deg_sc(dst_deg, ones_rows, zz)
    g, dinv = _tc_first(xp, W0, b0, Wc[0], degpart)
    for layer in range(1, NUM_LAYERS):
        agg = _agg_sc(g, s01, dm, zz).reshape(NP, HID)
        g = _tc_mid(agg, dinv, bc[layer - 1], Wc[layer])
    agg = _agg_sc(g, s01, dm, zz).reshape(NP, HID)
    logits = _tc_last(agg, dinv, bc[NUM_LAYERS - 1], Wl, bl)
    return logits[:N]


# speedup vs baseline: 1.1339x; 1.0011x over previous
import functools
import math

import jax
import jax.numpy as jnp
from jax import lax
from jax.experimental import pallas as pl
from jax.experimental.pallas import tpu as pltpu
from jax.experimental.pallas import tpu_sc as plsc

N = 10000
E = 320000
IN_CH = 128
HID = 128
OUT_CH = 40
NUM_LAYERS = 4

NC = 2
NS = 16
NW = NC * NS
NP = 10240
RPS = NP // NS
CH = 16
EPT = NP
CPT = EPT // CH
CPT2 = (NW * EPT) // (NS * CH)
PAD = EPT - E // NW
DCH = 64
DCPT = EPT // DCH
HR = NP // 2
HRS = HR // NS
ZROW = 10000
TROWS = 10008
TRS = 632
TLAST = N - (NS - 1) * TRS

BLK = 512
GRID = NP // BLK

_C = math.sqrt(math.pi) / 2.0

_mesh = plsc.VectorSubcoreMesh(core_axis_name="c", subcore_axis_name="s")


DEPTH = 3
IRING = 12
IP = 10
DDEPTH = 4


@functools.partial(
    pl.kernel,
    out_type=jax.ShapeDtypeStruct((NC, NP, HID), jnp.float32),
    mesh=_mesh,
    scratch_types=[
        pltpu.VMEM((DCPT, DCH), jnp.int32),
        pltpu.VMEM((DCH, HID), jnp.float32),
        pltpu.VMEM_SHARED((NP, HID), jnp.float32),
        pltpu.SemaphoreType.DMA((DDEPTH,)),
    ],
)
def _deg_sc(dst_hbm, ones_hbm, zz_hbm, out_hbm, dall, ones_v, dacc, sS):
    c = lax.axis_index("c")
    s = lax.axis_index("s")
    wid = s * NC + c
    pltpu.sync_copy(zz_hbm.at[pl.ds(s * RPS, RPS)], dacc.at[pl.ds(s * RPS, RPS)])
    pltpu.sync_copy(dst_hbm.at[pl.ds(wid * DCPT, DCPT)], dall)
    pltpu.sync_copy(ones_hbm, ones_v)
    plsc.subcore_barrier()

    def body(j, carry):
        q = lax.rem(j, DDEPTH)

        @pl.when(j >= DDEPTH)
        def _():
            pltpu.make_async_copy(ones_v, dacc.at[dall.at[j - DDEPTH]],
                                  sS.at[q]).wait()

        pltpu.async_copy(ones_v, dacc.at[dall.at[j]], sS.at[q], add=True)
        return carry

    lax.fori_loop(0, DCPT, body, 0)
    for k in range(DCPT - DDEPTH, DCPT):
        pltpu.make_async_copy(ones_v, dacc.at[dall.at[k]],
                              sS.at[k % DDEPTH]).wait()
    plsc.subcore_barrier()
    pltpu.sync_copy(dacc.at[pl.ds(s * RPS, RPS)],
                    out_hbm.at[c, pl.ds(s * RPS, RPS)])


@functools.partial(
    pl.kernel,
    out_type=jax.ShapeDtypeStruct((NC, HR, HID), jnp.float32),
    mesh=_mesh,
    scratch_types=[
        pltpu.VMEM((IRING, 1, CH), jnp.int32),
        pltpu.VMEM((IRING, 1, CH), jnp.int32),
        pltpu.VMEM((DEPTH, CH, HID), jnp.float32),
        pltpu.VMEM_SHARED((TROWS, HID), jnp.float32),
        pltpu.VMEM_SHARED((HR, HID), jnp.float32),
        pltpu.SemaphoreType.DMA((IRING,)),
        pltpu.SemaphoreType.DMA((IRING,)),
        pltpu.SemaphoreType.DMA((DEPTH,)),
        pltpu.SemaphoreType.DMA((DEPTH,)),
    ],
)
def _agg_sc(g_hbm, s_hbm, dm_hbm, zz_hbm, out_hbm,
            sring, dring, rbuf, table, acc, sI, sJ, sG, sS):
    c = lax.axis_index("c")
    s = lax.axis_index("s")

    @pl.when(s < NS - 1)
    def _():
        pltpu.sync_copy(g_hbm.at[pl.ds(s * TRS, TRS)],
                        table.at[pl.ds(s * TRS, TRS)])

    @pl.when(s == NS - 1)
    def _():
        pltpu.sync_copy(g_hbm.at[pl.ds((NS - 1) * TRS, TLAST)],
                        table.at[pl.ds((NS - 1) * TRS, TLAST)])

    @pl.when(s == 0)
    def _():
        pltpu.sync_copy(zz_hbm.at[pl.ds(0, TROWS - ZROW)],
                        table.at[pl.ds(ZROW, TROWS - ZROW)])

    pltpu.sync_copy(g_hbm.at[pl.ds(c * HR + s * HRS, HRS)],
                    acc.at[pl.ds(s * HRS, HRS)])
    plsc.subcore_barrier()

    for k in range(IP):
        pltpu.async_copy(s_hbm.at[c, s * CPT2 + k], sring.at[k], sI.at[k])
        pltpu.async_copy(dm_hbm.at[s * CPT2 + k], dring.at[k], sJ.at[k])
    for k in range(DEPTH - 1):
        pltpu.make_async_copy(s_hbm.at[c, s * CPT2 + k], sring.at[k],
                              sI.at[k]).wait()
        pltpu.async_copy(table.at[sring.at[k, 0]], rbuf.at[k], sG.at[k])

    def body(j, carry):
        q = lax.rem(j, DEPTH)
        r = lax.rem(j, IRING)
        pltpu.make_async_copy(table.at[sring.at[r, 0]], rbuf.at[q],
                              sG.at[q]).wait()
        pltpu.make_async_copy(dm_hbm.at[s * CPT2 + j], dring.at[r],
                              sJ.at[r]).wait()
        pltpu.async_copy(rbuf.at[q], acc.at[dring.at[r, 0]], sS.at[q],
                         add=True)

        @pl.when(j >= 1)
        def _():
            qs = lax.rem(j + DEPTH - 1, DEPTH)
            rs = lax.rem(j + IRING - 1, IRING)
            pltpu.make_async_copy(rbuf.at[qs], acc.at[dring.at[rs, 0]],
                                  sS.at[qs]).wait()

        @pl.when(j + DEPTH - 1 < CPT2)
        def _():
            qn = lax.rem(j + DEPTH - 1, DEPTH)
            rn = lax.rem(j + DEPTH - 1, IRING)
            pltpu.make_async_copy(s_hbm.at[c, s * CPT2 + j + DEPTH - 1],
                                  sring.at[rn], sI.at[rn]).wait()
            pltpu.async_copy(table.at[sring.at[rn, 0]], rbuf.at[qn],
                             sG.at[qn])

        @pl.when(j + IP < CPT2)
        def _():
            rp = lax.rem(j + IP, IRING)
            pltpu.async_copy(s_hbm.at[c, s * CPT2 + j + IP],
                             sring.at[rp], sI.at[rp])
            pltpu.async_copy(dm_hbm.at[s * CPT2 + j + IP],
                             dring.at[rp], sJ.at[rp])

        return carry

    lax.fori_loop(0, CPT2, body, 0)
    pltpu.make_async_copy(rbuf.at[(CPT2 - 1) % DEPTH],
                          acc.at[dring.at[(CPT2 - 1) % IRING, 0]],
                          sS.at[(CPT2 - 1) % DEPTH]).wait()
    plsc.subcore_barrier()
    pltpu.sync_copy(acc.at[pl.ds(s * HRS, HRS)],
                    out_hbm.at[c, pl.ds(s * HRS, HRS)])


def _erf(z):
    return lax.erf(z)


def _mm_t(a, b):
    return lax.dot_general(a, b, (((1,), (1,)), ((), ())),
                           preferred_element_type=jnp.float32)


def _tc_first_body(x_ref, w0_ref, b0_ref, wc_ref, deg_ref, g_ref, dinv_ref):
    x = x_ref[...]
    h = _erf(_C * (_mm_t(x, w0_ref[...]) + b0_ref[...]))
    deg = deg_ref[0, :, 0:1] + deg_ref[1, :, 0:1] + 1.0
    dinv = lax.rsqrt(deg)
    dinv_ref[...] = dinv
    g_ref[...] = dinv * _mm_t(h, wc_ref[...])


_tc_first = pl.pallas_call(
    _tc_first_body,
    grid=(GRID,),
    in_specs=[
        pl.BlockSpec((BLK, IN_CH), lambda i: (i, 0)),
        pl.BlockSpec((HID, IN_CH), lambda i: (0, 0)),
        pl.BlockSpec((HID,), lambda i: (0,)),
        pl.BlockSpec((HID, HID), lambda i: (0, 0)),
        pl.BlockSpec((NC, BLK, HID), lambda i: (0, i, 0)),
    ],
    out_specs=[
        pl.BlockSpec((BLK, HID), lambda i: (i, 0)),
        pl.BlockSpec((BLK, 1), lambda i: (i, 0)),
    ],
    out_shape=[
        jax.ShapeDtypeStruct((NP, HID), jnp.float32),
        jax.ShapeDtypeStruct((NP, 1), jnp.float32),
    ],
)


def _tc_mid_body(acc_ref, dinv_ref, b_ref, w_ref, out_ref):
    dinv = dinv_ref[...]
    h = _erf(_C * (dinv * acc_ref[...] + b_ref[...]))
    out_ref[...] = dinv * _mm_t(h, w_ref[...])


_tc_mid = pl.pallas_call(
    _tc_mid_body,
    grid=(GRID,),
    in_specs=[
        pl.BlockSpec((BLK, HID), lambda i: (i, 0)),
        pl.BlockSpec((BLK, 1), lambda i: (i, 0)),
        pl.BlockSpec((HID,), lambda i: (0,)),
        pl.BlockSpec((HID, HID), lambda i: (0, 0)),
    ],
    out_specs=pl.BlockSpec((BLK, HID), lambda i: (i, 0)),
    out_shape=jax.ShapeDtypeStruct((NP, HID), jnp.float32),
)


def _tc_last_body(acc_ref, dinv_ref, b_ref, wl_ref, bl_ref, out_ref):
    dinv = dinv_ref[...]
    h = _erf(_C * (dinv * acc_ref[...] + b_ref[...]))
    out_ref[...] = _mm_t(h, wl_ref[...]) + bl_ref[...]


_tc_last = pl.pallas_call(
    _tc_last_body,
    grid=(GRID,),
    in_specs=[
        pl.BlockSpec((BLK, HID), lambda i: (i, 0)),
        pl.BlockSpec((BLK, 1), lambda i: (i, 0)),
        pl.BlockSpec((HID,), lambda i: (0,)),
        pl.BlockSpec((OUT_CH, HID), lambda i: (0, 0)),
        pl.BlockSpec((OUT_CH,), lambda i: (0,)),
    ],
    out_specs=pl.BlockSpec((BLK, OUT_CH), lambda i: (i, 0)),
    out_shape=jax.ShapeDtypeStruct((NP, OUT_CH), jnp.float32),
)


def kernel(x, edge_index, W0, b0, Wc, bc, Wl, bl):
    xp = jnp.concatenate(
        [x, jnp.zeros((NP - N, IN_CH), jnp.float32)], axis=0)
    src = edge_index[0].reshape(NW, E // NW)
    dst = edge_index[1].reshape(NW, E // NW)
    pad_src = jnp.zeros((NW, PAD), jnp.int32)
    pad_dst = jnp.broadcast_to(N + jnp.arange(PAD, dtype=jnp.int32), (NW, PAD))
    srcp = jnp.concatenate([src, pad_src], axis=1).reshape(-1)
    dstp = jnp.concatenate([dst, pad_dst], axis=1).reshape(-1)

    in0 = dstp < HR
    s0 = jnp.where(in0, srcp, ZROW).reshape(NS * CPT2, 1, CH)
    s1 = jnp.where(in0, ZROW, srcp).reshape(NS * CPT2, 1, CH)
    s01 = jnp.stack([s0, s1])
    dm = jnp.where(in0, dstp, dstp - HR).reshape(NS * CPT2, 1, CH)

    dst_deg = dstp.reshape(NW * DCPT, DCH)
    ones_rows = jnp.ones((DCH, HID), jnp.float32)
    zz = jnp.zeros((NP, HID), jnp.float32)

    degpart = _deg_sc(dst_deg, ones_rows, zz)
    g, dinv = _tc_first(xp, W0, b0, Wc[0], degpart)
    for layer in range(1, NUM_LAYERS):
        agg = _agg_sc(g, s01, dm, zz).reshape(NP, HID)
        g = _tc_mid(agg, dinv, bc[layer - 1], Wc[layer])
    agg = _agg_sc(g, s01, dm, zz).reshape(NP, HID)
    logits = _tc_last(agg, dinv, bc[NUM_LAYERS - 1], Wl, bl)
    return logits[:N]
